# baseline (speedup 1.0000x reference)
import numpy as np
import jax
import jax.numpy as jnp
from jax.experimental import pallas as pl
from jax.experimental.pallas import tpu as pltpu

_PAD = 64
_B = 8
_TAPS9 = [(dh, dw) for dh in (-1, 0, 1) for dw in (-1, 0, 1)]
_BF = jnp.bfloat16


def _tap_masks_np(S, B):
    P = S * S
    m = np.zeros((9, 1, P), np.float32)
    for t, (dh, dw) in enumerate(_TAPS9):
        for h in range(S):
            for w in range(S):
                if 0 <= h + dh < S and 0 <= w + dw < S:
                    m[t, 0, h * S + w] = 1.0
    return np.tile(m, (1, 1, B))


def _pool_select_np(S, B):
    So = S // 2
    g = np.zeros((S * S, So * So), np.float32)
    for ho in range(So):
        for wo in range(So):
            g[(2 * ho) * S + 2 * wo, ho * So + wo] = 1.0
    return np.kron(np.eye(B, dtype=np.float32), g)


def _upsample_scatter_np(S, B):
    p = np.zeros((4, S * S, 4 * S * S), np.float32)
    for kh in range(2):
        for kw in range(2):
            t = kh * 2 + kw
            for h in range(S):
                for w in range(S):
                    p[t, h * S + w, (2 * h + kh) * (2 * S) + (2 * w + kw)] = 1.0
    return np.concatenate(
        [np.kron(np.eye(B, dtype=np.float32), p[t]) for t in range(4)], axis=0)


def _stage(pad, val):
    cin, L = val.shape
    pad[:cin, pl.ds(_PAD, L)] = val.astype(_BF)


def _conv3x3_relu(pad, cin, S, L, w_ref, b_ref, m_ref):
    taps = []
    for t, (dh, dw) in enumerate(_TAPS9):
        off = dh * S + dw
        taps.append(pad[:cin, pl.ds(_PAD + off, L)] * m_ref[t])
    big = jnp.concatenate(taps, axis=0)
    acc = jnp.dot(w_ref[...], big, preferred_element_type=jnp.float32)
    return jnp.maximum(acc + b_ref[...], 0.0)


def _maxpool2x2(pad, cin, S, L, g_ref):
    x0 = pad[:cin, pl.ds(_PAD, L)]
    t1 = pad[:cin, pl.ds(_PAD + 1, L)]
    t2 = pad[:cin, pl.ds(_PAD + S, L)]
    t3 = pad[:cin, pl.ds(_PAD + S + 1, L)]
    m = jnp.maximum(jnp.maximum(x0, t1), jnp.maximum(t2, t3))
    return jnp.dot(m, g_ref[...], preferred_element_type=jnp.float32)


def _conv_transpose2x2(xcat, w_ref, b_ref, p_ref):
    parts = [jnp.dot(w_ref[t], xcat, preferred_element_type=jnp.float32)
             for t in range(4)]
    alane = jnp.concatenate(parts, axis=1).astype(_BF)
    return jnp.dot(alane, p_ref[...], preferred_element_type=jnp.float32) + b_ref[...]


def _unet_kernel(x_ref, m16, m8, m4, m2,
                 w11, b11, w12, b12, w21, b21, w22, b22,
                 w31, b31, w32, b32, wb1, bb1, wb2, bb2,
                 g1, g2, g3, u3w, u3b, u2w, u2b, u1w, u1b,
                 p2, p4, p8, ow, ob, o_ref, pad_a, pad_b):
    pads = (pad_a, pad_b)
    CH = len(pads)
    C, P = x_ref.shape[1], x_ref.shape[2]
    c2, c4, c8 = C // 2, C // 4, C // 8
    L1, L2, L3, L4 = _B * P, _B * P // 4, _B * P // 16, _B * P // 64

    def both(f):
        return [f(i) for i in range(CH)]

    def stage_all(vals):
        for i in range(CH):
            _stage(pads[i], vals[i])

    for i in range(CH):
        pads[i][...] = jnp.zeros_like(pads[i])
    for i in range(CH):
        for b in range(_B):
            pads[i][:C, pl.ds(_PAD + b * P, P)] = x_ref[i * _B + b].astype(_BF)

    stage_all(both(lambda i: _conv3x3_relu(pads[i], C, 16, L1, w11, b11, m16)))
    e1b = [v.astype(_BF)
           for v in both(lambda i: _conv3x3_relu(pads[i], c2, 16, L1, w12, b12, m16))]
    stage_all(e1b)
    stage_all(both(lambda i: _maxpool2x2(pads[i], c2, 16, L1, g1)))
    stage_all(both(lambda i: _conv3x3_relu(pads[i], c2, 8, L2, w21, b21, m8)))
    e2b = [v.astype(_BF)
           for v in both(lambda i: _conv3x3_relu(pads[i], c4, 8, L2, w22, b22, m8))]
    stage_all(e2b)
    stage_all(both(lambda i: _maxpool2x2(pads[i], c4, 8, L2, g2)))
    stage_all(both(lambda i: _conv3x3_relu(pads[i], c4, 4, L3, w31, b31, m4)))
    e3b = [v.astype(_BF)
           for v in both(lambda i: _conv3x3_relu(pads[i], c8, 4, L3, w32, b32, m4))]
    stage_all(e3b)
    stage_all(both(lambda i: _maxpool2x2(pads[i], c8, 4, L3, g3)))
    stage_all(both(lambda i: _conv3x3_relu(pads[i], c8, 2, L4, wb1, bb1, m2)))
    bn = both(lambda i: _conv3x3_relu(pads[i], c8, 2, L4, wb2, bb2, m2))

    u3 = both(lambda i: _conv_transpose2x2(bn[i].astype(_BF), u3w, u3b, p2))
    u2 = both(lambda i: _conv_transpose2x2(
        jnp.concatenate([u3[i].astype(_BF), e3b[i]], axis=0), u2w, u2b, p4))
    u1 = both(lambda i: _conv_transpose2x2(
        jnp.concatenate([u2[i].astype(_BF), e2b[i]], axis=0), u1w, u1b, p8))
    out = both(lambda i: jnp.dot(
        ow[...], jnp.concatenate([u1[i].astype(_BF), e1b[i]], axis=0),
        preferred_element_type=jnp.float32) + ob[...])
    for i in range(CH):
        for b in range(_B):
            o_ref[i * _B + b] = out[i][:, b * P:(b + 1) * P]


def _flat9(w):
    return jnp.transpose(w, (3, 0, 1, 2)).reshape(w.shape[3], -1).astype(_BF)


def _t4(w):
    return jnp.transpose(w, (0, 1, 3, 2)).reshape(4, w.shape[3], w.shape[2]).astype(_BF)


def _col(b):
    return b.reshape(-1, 1)


def kernel(enc1_w1, enc1_b1, enc1_w2, enc1_b2, enc2_w1, enc2_b1, enc2_w2,
           enc2_b2, enc3_w1, enc3_b1, enc3_w2, enc3_b2, bn_w1, bn_b1, bn_w2,
           bn_b2, up3_w, up3_b, up2_w, up2_b, up1_w, up1_b, out_w, out_b, x):
    N, C, H, W = x.shape
    P = H * W
    B = _B
    bf = lambda a: jnp.asarray(a, dtype=_BF)

    consts = (
        bf(_tap_masks_np(16, B)), bf(_tap_masks_np(8, B)),
        bf(_tap_masks_np(4, B)), bf(_tap_masks_np(2, B)),
        _flat9(enc1_w1), _col(enc1_b1), _flat9(enc1_w2), _col(enc1_b2),
        _flat9(enc2_w1), _col(enc2_b1), _flat9(enc2_w2), _col(enc2_b2),
        _flat9(enc3_w1), _col(enc3_b1), _flat9(enc3_w2), _col(enc3_b2),
        _flat9(bn_w1), _col(bn_b1), _flat9(bn_w2), _col(bn_b2),
        bf(_pool_select_np(16, B)), bf(_pool_select_np(8, B)),
        bf(_pool_select_np(4, B)),
        _t4(up3_w), _col(up3_b), _t4(up2_w), _col(up2_b), _t4(up1_w), _col(up1_b),
        bf(_upsample_scatter_np(2, B)), bf(_upsample_scatter_np(4, B)),
        bf(_upsample_scatter_np(8, B)),
        out_w.T.astype(_BF), _col(out_b),
    )

    x2 = x.reshape(N, C, P)
    G = 2 * B
    pad_lanes = (_PAD + B * P + _PAD + 127) // 128 * 128

    def _call(xs, *cs):
        Ns = xs.shape[0]
        in_specs = [pl.BlockSpec((G, C, P), lambda n: (n, 0, 0))]
        for a in cs:
            in_specs.append(pl.BlockSpec(a.shape, lambda n, _nd=a.ndim: (0,) * _nd))
        return pl.pallas_call(
            _unet_kernel,
            out_shape=jax.ShapeDtypeStruct((Ns, C, P), jnp.float32),
            grid=(Ns // G,),
            in_specs=in_specs,
            out_specs=pl.BlockSpec((G, C, P), lambda n: (n, 0, 0)),
            scratch_shapes=[pltpu.VMEM((C, pad_lanes), _BF),
                            pltpu.VMEM((C, pad_lanes), _BF)],
            compiler_params=pltpu.CompilerParams(
                dimension_semantics=("parallel",),
                vmem_limit_bytes=64 * 1024 * 1024),
        )(xs, *cs)

    devs = jax.devices()
    if len(devs) >= 2 and (N // 2) % G == 0:
        from jax.experimental.shard_map import shard_map
        mesh = jax.sharding.Mesh(np.array(devs[:2]), ("d",))
        ps = jax.sharding.PartitionSpec
        out = shard_map(
            _call, mesh=mesh,
            in_specs=(ps("d"),) + (ps(),) * len(consts),
            out_specs=ps("d"), check_rep=False)(x2, *consts)
    else:
        out = _call(x2, *consts)
    return out.reshape(N, C, H, W)


# speedup vs baseline: 2.4387x; 2.4387x over previous
import numpy as np
import jax
import jax.numpy as jnp
from jax.experimental import pallas as pl
from jax.experimental.pallas import tpu as pltpu

_PAD = 64
_B = 8
_CH = 4
_TAPS9 = [(dh, dw) for dh in (-1, 0, 1) for dw in (-1, 0, 1)]
_BF = jnp.bfloat16


def _tap_masks_np(S, B):
    P = S * S
    m = np.zeros((9, 1, P), np.float32)
    for t, (dh, dw) in enumerate(_TAPS9):
        for h in range(S):
            for w in range(S):
                if 0 <= h + dh < S and 0 <= w + dw < S:
                    m[t, 0, h * S + w] = 1.0
    return np.tile(m, (1, 1, B))


def _pool_select_np(S, B):
    So = S // 2
    g = np.zeros((S * S, So * So), np.float32)
    for ho in range(So):
        for wo in range(So):
            g[(2 * ho) * S + 2 * wo, ho * So + wo] = 1.0
    return np.kron(np.eye(B, dtype=np.float32), g)


def _upsample_scatter_np(S, B):
    p = np.zeros((4, S * S, 4 * S * S), np.float32)
    for kh in range(2):
        for kw in range(2):
            t = kh * 2 + kw
            for h in range(S):
                for w in range(S):
                    p[t, h * S + w, (2 * h + kh) * (2 * S) + (2 * w + kw)] = 1.0
    return np.concatenate(
        [np.kron(np.eye(B, dtype=np.float32), p[t]) for t in range(4)], axis=0)


def _stage(pad, val):
    cin, L = val.shape
    pad[:cin, pl.ds(_PAD, L)] = val.astype(_BF)


def _conv3x3_relu(pad, cin, S, L, w_ref, b_ref, m_ref):
    taps = []
    for t, (dh, dw) in enumerate(_TAPS9):
        off = dh * S + dw
        taps.append(pad[:cin, pl.ds(_PAD + off, L)] * m_ref[t])
    big = jnp.concatenate(taps, axis=0)
    acc = jnp.dot(w_ref[...], big, preferred_element_type=jnp.float32)
    return jnp.maximum(acc + b_ref[...], 0.0)


def _maxpool2x2(pad, cin, S, L, g_ref):
    x0 = pad[:cin, pl.ds(_PAD, L)]
    t1 = pad[:cin, pl.ds(_PAD + 1, L)]
    t2 = pad[:cin, pl.ds(_PAD + S, L)]
    t3 = pad[:cin, pl.ds(_PAD + S + 1, L)]
    m = jnp.maximum(jnp.maximum(x0, t1), jnp.maximum(t2, t3))
    return jnp.dot(m, g_ref[...], preferred_element_type=jnp.float32)


def _conv_transpose2x2(xcat, w_ref, b_ref, p_ref):
    parts = [jnp.dot(w_ref[t], xcat, preferred_element_type=jnp.float32)
             for t in range(4)]
    alane = jnp.concatenate(parts, axis=1).astype(_BF)
    return jnp.dot(alane, p_ref[...], preferred_element_type=jnp.float32) + b_ref[...]


def _unet_kernel(x_ref, m16, m8, m4, m2,
                 w11, b11, w12, b12, w21, b21, w22, b22,
                 w31, b31, w32, b32, wb1, bb1, wb2, bb2,
                 g1, g2, g3, u3w, u3b, u2w, u2b, u1w, u1b,
                 p2, p4, p8, ow, ob, o_ref, *pads):
    CH = len(pads)
    C, P = x_ref.shape[1], x_ref.shape[2]
    c2, c4, c8 = C // 2, C // 4, C // 8
    L1, L2, L3, L4 = _B * P, _B * P // 4, _B * P // 16, _B * P // 64

    def both(f):
        return [f(i) for i in range(CH)]

    def stage_all(vals):
        for i in range(CH):
            _stage(pads[i], vals[i])

    for i in range(CH):
        pads[i][...] = jnp.zeros_like(pads[i])
    for i in range(CH):
        for b in range(_B):
            pads[i][:C, pl.ds(_PAD + b * P, P)] = x_ref[i * _B + b].astype(_BF)

    stage_all(both(lambda i: _conv3x3_relu(pads[i], C, 16, L1, w11, b11, m16)))
    e1b = [v.astype(_BF)
           for v in both(lambda i: _conv3x3_relu(pads[i], c2, 16, L1, w12, b12, m16))]
    stage_all(e1b)
    stage_all(both(lambda i: _maxpool2x2(pads[i], c2, 16, L1, g1)))
    stage_all(both(lambda i: _conv3x3_relu(pads[i], c2, 8, L2, w21, b21, m8)))
    e2b = [v.astype(_BF)
           for v in both(lambda i: _conv3x3_relu(pads[i], c4, 8, L2, w22, b22, m8))]
    stage_all(e2b)
    stage_all(both(lambda i: _maxpool2x2(pads[i], c4, 8, L2, g2)))
    stage_all(both(lambda i: _conv3x3_relu(pads[i], c4, 4, L3, w31, b31, m4)))
    e3b = [v.astype(_BF)
           for v in both(lambda i: _conv3x3_relu(pads[i], c8, 4, L3, w32, b32, m4))]
    stage_all(e3b)
    stage_all(both(lambda i: _maxpool2x2(pads[i], c8, 4, L3, g3)))
    stage_all(both(lambda i: _conv3x3_relu(pads[i], c8, 2, L4, wb1, bb1, m2)))
    bn = both(lambda i: _conv3x3_relu(pads[i], c8, 2, L4, wb2, bb2, m2))

    u3 = both(lambda i: _conv_transpose2x2(bn[i].astype(_BF), u3w, u3b, p2))
    u2 = both(lambda i: _conv_transpose2x2(
        jnp.concatenate([u3[i].astype(_BF), e3b[i]], axis=0), u2w, u2b, p4))
    u1 = both(lambda i: _conv_transpose2x2(
        jnp.concatenate([u2[i].astype(_BF), e2b[i]], axis=0), u1w, u1b, p8))
    out = both(lambda i: jnp.dot(
        ow[...], jnp.concatenate([u1[i].astype(_BF), e1b[i]], axis=0),
        preferred_element_type=jnp.float32) + ob[...])
    for i in range(CH):
        for b in range(_B):
            o_ref[i * _B + b] = out[i][:, b * P:(b + 1) * P]


def _flat9(w):
    return jnp.transpose(w, (3, 0, 1, 2)).reshape(w.shape[3], -1).astype(_BF)


def _t4(w):
    return jnp.transpose(w, (0, 1, 3, 2)).reshape(4, w.shape[3], w.shape[2]).astype(_BF)


def _col(b):
    return b.reshape(-1, 1)


def kernel(enc1_w1, enc1_b1, enc1_w2, enc1_b2, enc2_w1, enc2_b1, enc2_w2,
           enc2_b2, enc3_w1, enc3_b1, enc3_w2, enc3_b2, bn_w1, bn_b1, bn_w2,
           bn_b2, up3_w, up3_b, up2_w, up2_b, up1_w, up1_b, out_w, out_b, x):
    N, C, H, W = x.shape
    P = H * W
    B = _B
    bf = lambda a: jnp.asarray(a, dtype=_BF)

    consts = (
        bf(_tap_masks_np(16, B)), bf(_tap_masks_np(8, B)),
        bf(_tap_masks_np(4, B)), bf(_tap_masks_np(2, B)),
        _flat9(enc1_w1), _col(enc1_b1), _flat9(enc1_w2), _col(enc1_b2),
        _flat9(enc2_w1), _col(enc2_b1), _flat9(enc2_w2), _col(enc2_b2),
        _flat9(enc3_w1), _col(enc3_b1), _flat9(enc3_w2), _col(enc3_b2),
        _flat9(bn_w1), _col(bn_b1), _flat9(bn_w2), _col(bn_b2),
        bf(_pool_select_np(16, B)), bf(_pool_select_np(8, B)),
        bf(_pool_select_np(4, B)),
        _t4(up3_w), _col(up3_b), _t4(up2_w), _col(up2_b), _t4(up1_w), _col(up1_b),
        bf(_upsample_scatter_np(2, B)), bf(_upsample_scatter_np(4, B)),
        bf(_upsample_scatter_np(8, B)),
        out_w.T.astype(_BF), _col(out_b),
    )

    x2 = x.reshape(N, C, P)
    G = _CH * B
    pad_lanes = (_PAD + B * P + _PAD + 127) // 128 * 128

    def _call(xs, *cs):
        Ns = xs.shape[0]
        in_specs = [pl.BlockSpec((G, C, P), lambda n: (n, 0, 0))]
        for a in cs:
            in_specs.append(pl.BlockSpec(a.shape, lambda n, _nd=a.ndim: (0,) * _nd))
        return pl.pallas_call(
            _unet_kernel,
            out_shape=jax.ShapeDtypeStruct((Ns, C, P), jnp.float32),
            grid=(Ns // G,),
            in_specs=in_specs,
            out_specs=pl.BlockSpec((G, C, P), lambda n: (n, 0, 0)),
            scratch_shapes=[pltpu.VMEM((C, pad_lanes), _BF)
                            for _ in range(_CH)],
            compiler_params=pltpu.CompilerParams(
                dimension_semantics=("parallel",),
                vmem_limit_bytes=64 * 1024 * 1024),
        )(xs, *cs)

    out = _call(x2, *consts)
    return out.reshape(N, C, H, W)


# speedup vs baseline: 2.9246x; 1.1992x over previous
import numpy as np
import jax
import jax.numpy as jnp
from jax.experimental import pallas as pl
from jax.experimental.pallas import tpu as pltpu

_B = 8
_CH = 4
_TAPS9 = [(dh, dw) for dh in (-1, 0, 1) for dw in (-1, 0, 1)]
_BF = jnp.bfloat16
_RSTRIDE = _B * 256 + 256
_R0, _R1, _R2 = 128, 128 + _RSTRIDE, 128 + 2 * _RSTRIDE
_PAD = _R0


def _tap_masks_np(S, B):
    P = S * S
    m = np.zeros((4, 1, P), np.float32)
    for h in range(S):
        for w in range(S):
            l = h * S + w
            m[0, 0, l] = 1.0 if w - 1 >= 0 else 0.0
            m[1, 0, l] = 1.0 if w + 1 < S else 0.0
            m[2, 0, l] = 1.0 if h - 1 >= 0 else 0.0
            m[3, 0, l] = 1.0 if h + 1 < S else 0.0
    return np.tile(m, (1, 1, B))


def _pool_select_np(S, B):
    So = S // 2
    g = np.zeros((S * S, So * So), np.float32)
    for ho in range(So):
        for wo in range(So):
            g[(2 * ho) * S + 2 * wo, ho * So + wo] = 1.0
    return np.kron(np.eye(B, dtype=np.float32), g)


def _upsample_scatter_np(S, B):
    p = np.zeros((4, S * S, 4 * S * S), np.float32)
    for kh in range(2):
        for kw in range(2):
            t = kh * 2 + kw
            for h in range(S):
                for w in range(S):
                    p[t, h * S + w, (2 * h + kh) * (2 * S) + (2 * w + kw)] = 1.0
    return np.concatenate(
        [np.kron(np.eye(B, dtype=np.float32), p[t]) for t in range(4)], axis=0)


def _stage(pad, val):
    cin, L = val.shape
    pad[:cin, pl.ds(_PAD, L)] = val.astype(_BF)


def _conv3x3_relu(pad, cin, S, L, w_ref, b_ref, m_ref):
    pad[:cin, pl.ds(_R1, L)] = pad[:cin, pl.ds(_R0 - 1, L)] * m_ref[0]
    pad[:cin, pl.ds(_R2, L)] = pad[:cin, pl.ds(_R0 + 1, L)] * m_ref[1]
    taps = []
    for t, (dh, dw) in enumerate(_TAPS9):
        base = _R0 if dw == 0 else (_R1 if dw == -1 else _R2)
        v = pad[:cin, pl.ds(base + dh * S, L)]
        if dh == -1:
            v = v * m_ref[2]
        elif dh == 1:
            v = v * m_ref[3]
        taps.append(v)
    big = jnp.concatenate(taps, axis=0)
    acc = jnp.dot(w_ref[...], big, preferred_element_type=jnp.float32)
    return jnp.maximum(acc + b_ref[...], 0.0)


def _maxpool2x2(pad, cin, S, L, g_ref):
    x0 = pad[:cin, pl.ds(_PAD, L)]
    t1 = pad[:cin, pl.ds(_PAD + 1, L)]
    t2 = pad[:cin, pl.ds(_PAD + S, L)]
    t3 = pad[:cin, pl.ds(_PAD + S + 1, L)]
    m = jnp.maximum(jnp.maximum(x0, t1), jnp.maximum(t2, t3))
    return jnp.dot(m, g_ref[...], preferred_element_type=jnp.float32)


def _conv_transpose2x2(xcat, w_ref, b_ref, p_ref, nchunk=1):
    parts = [jnp.dot(w_ref[t], xcat, preferred_element_type=jnp.float32)
             for t in range(4)]
    Lin = xcat.shape[1]
    w = Lin // nchunk
    outs = []
    for c in range(nchunk):
        al = jnp.concatenate([pt[:, c * w:(c + 1) * w] for pt in parts],
                             axis=1).astype(_BF)
        outs.append(jnp.dot(al, p_ref[...], preferred_element_type=jnp.float32))
    o = outs[0] if nchunk == 1 else jnp.concatenate(outs, axis=1)
    return o + b_ref[...]


def _unet_kernel(x_ref, m16, m8, m4, m2,
                 w11, b11, w12, b12, w21, b21, w22, b22,
                 w31, b31, w32, b32, wb1, bb1, wb2, bb2,
                 g1, g2, g3, u3w, u3b, u2w, u2b, u1w, u1b,
                 p2, p4, p8, ow, ob, o_ref, *pads):
    CH = len(pads)
    C, P = x_ref.shape[1], x_ref.shape[2]
    c2, c4, c8 = C // 2, C // 4, C // 8
    L1, L2, L3, L4 = _B * P, _B * P // 4, _B * P // 16, _B * P // 64

    def both(f):
        return [f(i) for i in range(CH)]

    def stage_all(vals):
        for i in range(CH):
            _stage(pads[i], vals[i])

    for i in range(CH):
        pads[i][...] = jnp.zeros_like(pads[i])
    for i in range(CH):
        for b in range(_B):
            pads[i][:C, pl.ds(_PAD + b * P, P)] = x_ref[i * _B + b].astype(_BF)

    stage_all(both(lambda i: _conv3x3_relu(pads[i], C, 16, L1, w11, b11, m16)))
    e1b = [v.astype(_BF)
           for v in both(lambda i: _conv3x3_relu(pads[i], c2, 16, L1, w12, b12, m16))]
    stage_all(e1b)
    stage_all(both(lambda i: _maxpool2x2(pads[i], c2, 16, L1, g1)))
    stage_all(both(lambda i: _conv3x3_relu(pads[i], c2, 8, L2, w21, b21, m8)))
    e2b = [v.astype(_BF)
           for v in both(lambda i: _conv3x3_relu(pads[i], c4, 8, L2, w22, b22, m8))]
    stage_all(e2b)
    stage_all(both(lambda i: _maxpool2x2(pads[i], c4, 8, L2, g2)))
    stage_all(both(lambda i: _conv3x3_relu(pads[i], c4, 4, L3, w31, b31, m4)))
    e3b = [v.astype(_BF)
           for v in both(lambda i: _conv3x3_relu(pads[i], c8, 4, L3, w32, b32, m4))]
    stage_all(e3b)
    stage_all(both(lambda i: _maxpool2x2(pads[i], c8, 4, L3, g3)))
    stage_all(both(lambda i: _conv3x3_relu(pads[i], c8, 2, L4, wb1, bb1, m2)))
    bn = both(lambda i: _conv3x3_relu(pads[i], c8, 2, L4, wb2, bb2, m2))

    u3 = both(lambda i: _conv_transpose2x2(bn[i].astype(_BF), u3w, u3b, p2))
    u2 = both(lambda i: _conv_transpose2x2(
        jnp.concatenate([u3[i].astype(_BF), e3b[i]], axis=0), u2w, u2b, p4))
    u1 = both(lambda i: _conv_transpose2x2(
        jnp.concatenate([u2[i].astype(_BF), e2b[i]], axis=0), u1w, u1b, p8,
        nchunk=4))
    out = both(lambda i: jnp.dot(
        ow[...], jnp.concatenate([u1[i].astype(_BF), e1b[i]], axis=0),
        preferred_element_type=jnp.float32) + ob[...])
    for i in range(CH):
        for b in range(_B):
            o_ref[i * _B + b] = out[i][:, b * P:(b + 1) * P]


def _flat9(w):
    return jnp.transpose(w, (3, 0, 1, 2)).reshape(w.shape[3], -1).astype(_BF)


def _t4(w):
    return jnp.transpose(w, (0, 1, 3, 2)).reshape(4, w.shape[3], w.shape[2]).astype(_BF)


def _col(b):
    return b.reshape(-1, 1)


def kernel(enc1_w1, enc1_b1, enc1_w2, enc1_b2, enc2_w1, enc2_b1, enc2_w2,
           enc2_b2, enc3_w1, enc3_b1, enc3_w2, enc3_b2, bn_w1, bn_b1, bn_w2,
           bn_b2, up3_w, up3_b, up2_w, up2_b, up1_w, up1_b, out_w, out_b, x):
    N, C, H, W = x.shape
    P = H * W
    B = _B
    bf = lambda a: jnp.asarray(a, dtype=_BF)

    consts = (
        bf(_tap_masks_np(16, B)), bf(_tap_masks_np(8, B)),
        bf(_tap_masks_np(4, B)), bf(_tap_masks_np(2, B)),
        _flat9(enc1_w1), _col(enc1_b1), _flat9(enc1_w2), _col(enc1_b2),
        _flat9(enc2_w1), _col(enc2_b1), _flat9(enc2_w2), _col(enc2_b2),
        _flat9(enc3_w1), _col(enc3_b1), _flat9(enc3_w2), _col(enc3_b2),
        _flat9(bn_w1), _col(bn_b1), _flat9(bn_w2), _col(bn_b2),
        bf(_pool_select_np(16, B)), bf(_pool_select_np(8, B)),
        bf(_pool_select_np(4, B)),
        _t4(up3_w), _col(up3_b), _t4(up2_w), _col(up2_b), _t4(up1_w), _col(up1_b),
        bf(_upsample_scatter_np(2, B)), bf(_upsample_scatter_np(4, B)),
        bf(_upsample_scatter_np(8, 2)),
        out_w.T.astype(_BF), _col(out_b),
    )

    x2 = x.reshape(N, C, P)
    G = _CH * B
    pad_lanes = 128 + 3 * _RSTRIDE

    def _call(xs, *cs):
        Ns = xs.shape[0]
        in_specs = [pl.BlockSpec((G, C, P), lambda n: (n, 0, 0))]
        for a in cs:
            in_specs.append(pl.BlockSpec(a.shape, lambda n, _nd=a.ndim: (0,) * _nd))
        return pl.pallas_call(
            _unet_kernel,
            out_shape=jax.ShapeDtypeStruct((Ns, C, P), jnp.float32),
            grid=(Ns // G,),
            in_specs=in_specs,
            out_specs=pl.BlockSpec((G, C, P), lambda n: (n, 0, 0)),
            scratch_shapes=[pltpu.VMEM((C, pad_lanes), _BF)
                            for _ in range(_CH)],
            compiler_params=pltpu.CompilerParams(
                dimension_semantics=("parallel",),
                vmem_limit_bytes=64 * 1024 * 1024),
        )(xs, *cs)

    out = _call(x2, *consts)
    return out.reshape(N, C, H, W)


# speedup vs baseline: 4.0638x; 1.3896x over previous
import numpy as np
import jax
import jax.numpy as jnp
from jax.experimental import pallas as pl
from jax.experimental.pallas import tpu as pltpu

_B = 8
_CH = 4
_TAPS9 = [(dh, dw) for dh in (-1, 0, 1) for dw in (-1, 0, 1)]
_BF = jnp.bfloat16


def _tap_masks_np(S, B):
    P = S * S
    m = np.zeros((9, 1, P), np.float32)
    for t, (dh, dw) in enumerate(_TAPS9):
        for h in range(S):
            for w in range(S):
                if 0 <= h + dh < S and 0 <= w + dw < S:
                    m[t, 0, h * S + w] = 1.0
    return np.tile(m, (1, 1, B))


def _pool_select_np(S, B):
    So = S // 2
    g = np.zeros((S * S, So * So), np.float32)
    for ho in range(So):
        for wo in range(So):
            g[(2 * ho) * S + 2 * wo, ho * So + wo] = 1.0
    return np.kron(np.eye(B, dtype=np.float32), g)


def _upsample_scatter_np(S, B):
    p = np.zeros((4, S * S, 4 * S * S), np.float32)
    for kh in range(2):
        for kw in range(2):
            t = kh * 2 + kw
            for h in range(S):
                for w in range(S):
                    p[t, h * S + w, (2 * h + kh) * (2 * S) + (2 * w + kw)] = 1.0
    return np.concatenate(
        [np.kron(np.eye(B, dtype=np.float32), p[t]) for t in range(4)], axis=0)


def _conv3x3_relu(xb, S, w_ref, b_ref, m_ref):
    y = jnp.dot(w_ref[...], xb, preferred_element_type=jnp.float32)
    cout = y.shape[0] // 9
    acc = None
    for t, (dh, dw) in enumerate(_TAPS9):
        off = dh * S + dw
        s = y[t * cout:(t + 1) * cout]
        if off:
            s = jnp.roll(s, -off, axis=1)
        if dh != 0 or dw != 0:
            s = s * m_ref[t]
        acc = s if acc is None else acc + s
    return jnp.maximum(acc + b_ref[...], 0.0)


def _maxpool2x2(e, S, g_ref):
    t1 = jnp.roll(e, -1, axis=1)
    t2 = jnp.roll(e, -S, axis=1)
    t3 = jnp.roll(e, -(S + 1), axis=1)
    m = jnp.maximum(jnp.maximum(e, t1), jnp.maximum(t2, t3))
    return jnp.dot(m.astype(_BF), g_ref[...],
                   preferred_element_type=jnp.float32)


def _conv_transpose2x2(xcat, w_ref, b_ref, p_ref, nchunk=1):
    parts = [jnp.dot(w_ref[t], xcat, preferred_element_type=jnp.float32)
             for t in range(4)]
    Lin = xcat.shape[1]
    w = Lin // nchunk
    outs = []
    for c in range(nchunk):
        al = jnp.concatenate([pt[:, c * w:(c + 1) * w] for pt in parts],
                             axis=1).astype(_BF)
        outs.append(jnp.dot(al, p_ref[...], preferred_element_type=jnp.float32))
    o = outs[0] if nchunk == 1 else jnp.concatenate(outs, axis=1)
    return o + b_ref[...]


def _unet_kernel(x_ref, m16, m8, m4, m2,
                 w11, b11, w12, b12, w21, b21, w22, b22,
                 w31, b31, w32, b32, wb1, bb1, wb2, bb2,
                 g1, g2, g3, u3w, u3b, u2w, u2b, u1w, u1b,
                 p2, p4, p8, ow, ob, o_ref):
    C, P = x_ref.shape[1], x_ref.shape[2]

    def both(f):
        return [f(i) for i in range(_CH)]

    x = both(lambda i: jnp.concatenate(
        [x_ref[i * _B + b].astype(_BF) for b in range(_B)], axis=1))

    t = both(lambda i: _conv3x3_relu(x[i], 16, w11, b11, m16).astype(_BF))
    e1 = both(lambda i: _conv3x3_relu(t[i], 16, w12, b12, m16))
    e1b = [v.astype(_BF) for v in e1]
    p1 = both(lambda i: _maxpool2x2(e1[i], 16, g1).astype(_BF))
    t = both(lambda i: _conv3x3_relu(p1[i], 8, w21, b21, m8).astype(_BF))
    e2 = both(lambda i: _conv3x3_relu(t[i], 8, w22, b22, m8))
    e2b = [v.astype(_BF) for v in e2]
    p2v = both(lambda i: _maxpool2x2(e2[i], 8, g2).astype(_BF))
    t = both(lambda i: _conv3x3_relu(p2v[i], 4, w31, b31, m4).astype(_BF))
    e3 = both(lambda i: _conv3x3_relu(t[i], 4, w32, b32, m4))
    e3b = [v.astype(_BF) for v in e3]
    p3v = both(lambda i: _maxpool2x2(e3[i], 4, g3).astype(_BF))
    t = both(lambda i: _conv3x3_relu(p3v[i], 2, wb1, bb1, m2).astype(_BF))
    bn = both(lambda i: _conv3x3_relu(t[i], 2, wb2, bb2, m2))

    u3 = both(lambda i: _conv_transpose2x2(bn[i].astype(_BF), u3w, u3b, p2))
    u2 = both(lambda i: _conv_transpose2x2(
        jnp.concatenate([u3[i].astype(_BF), e3b[i]], axis=0), u2w, u2b, p4))
    u1 = both(lambda i: _conv_transpose2x2(
        jnp.concatenate([u2[i].astype(_BF), e2b[i]], axis=0), u1w, u1b, p8,
        nchunk=4))
    out = both(lambda i: jnp.dot(
        ow[...], jnp.concatenate([u1[i].astype(_BF), e1b[i]], axis=0),
        preferred_element_type=jnp.float32) + ob[...])
    for i in range(_CH):
        for b in range(_B):
            o_ref[i * _B + b] = out[i][:, b * P:(b + 1) * P]


def _w9(w):
    cin, cout = w.shape[2], w.shape[3]
    return jnp.transpose(w, (0, 1, 3, 2)).reshape(9 * cout, cin).astype(_BF)


def _t4(w):
    return jnp.transpose(w, (0, 1, 3, 2)).reshape(4, w.shape[3], w.shape[2]).astype(_BF)


def _col(b):
    return b.reshape(-1, 1)


def kernel(enc1_w1, enc1_b1, enc1_w2, enc1_b2, enc2_w1, enc2_b1, enc2_w2,
           enc2_b2, enc3_w1, enc3_b1, enc3_w2, enc3_b2, bn_w1, bn_b1, bn_w2,
           bn_b2, up3_w, up3_b, up2_w, up2_b, up1_w, up1_b, out_w, out_b, x):
    N, C, H, W = x.shape
    P = H * W
    B = _B
    f32a = lambda a: jnp.asarray(a, dtype=jnp.float32)
    bf = lambda a: jnp.asarray(a, dtype=_BF)

    consts = (
        f32a(_tap_masks_np(16, B)), f32a(_tap_masks_np(8, B)),
        f32a(_tap_masks_np(4, B)), f32a(_tap_masks_np(2, B)),
        _w9(enc1_w1), _col(enc1_b1), _w9(enc1_w2), _col(enc1_b2),
        _w9(enc2_w1), _col(enc2_b1), _w9(enc2_w2), _col(enc2_b2),
        _w9(enc3_w1), _col(enc3_b1), _w9(enc3_w2), _col(enc3_b2),
        _w9(bn_w1), _col(bn_b1), _w9(bn_w2), _col(bn_b2),
        bf(_pool_select_np(16, B)), bf(_pool_select_np(8, B)),
        bf(_pool_select_np(4, B)),
        _t4(up3_w), _col(up3_b), _t4(up2_w), _col(up2_b), _t4(up1_w), _col(up1_b),
        bf(_upsample_scatter_np(2, B)), bf(_upsample_scatter_np(4, B)),
        bf(_upsample_scatter_np(8, 2)),
        out_w.T.astype(_BF), _col(out_b),
    )

    x2 = x.reshape(N, C, P)
    G = _CH * B

    def _call(xs, *cs):
        Ns = xs.shape[0]
        in_specs = [pl.BlockSpec((G, C, P), lambda n: (n, 0, 0))]
        for a in cs:
            in_specs.append(pl.BlockSpec(a.shape, lambda n, _nd=a.ndim: (0,) * _nd))
        return pl.pallas_call(
            _unet_kernel,
            out_shape=jax.ShapeDtypeStruct((Ns, C, P), jnp.float32),
            grid=(Ns // G,),
            in_specs=in_specs,
            out_specs=pl.BlockSpec((G, C, P), lambda n: (n, 0, 0)),
            compiler_params=pltpu.CompilerParams(
                dimension_semantics=("parallel",),
                vmem_limit_bytes=64 * 1024 * 1024),
        )(xs, *cs)

    out = _call(x2, *consts)
    return out.reshape(N, C, H, W)


# speedup vs baseline: 4.3216x; 1.0634x over previous
import numpy as np
import jax
import jax.numpy as jnp
from jax.experimental import pallas as pl
from jax.experimental.pallas import tpu as pltpu

_B = 8
_CH = 8
_TAPS9 = [(dh, dw) for dh in (-1, 0, 1) for dw in (-1, 0, 1)]
_BF = jnp.bfloat16


def _tap_masks_np(S, B):
    P = S * S
    m = np.zeros((9, 1, P), np.float32)
    for t, (dh, dw) in enumerate(_TAPS9):
        for h in range(S):
            for w in range(S):
                if 0 <= h + dh < S and 0 <= w + dw < S:
                    m[t, 0, h * S + w] = 1.0
    return np.tile(m, (1, 1, B))


def _pool_select_np(S, B):
    So = S // 2
    g = np.zeros((S * S, So * So), np.float32)
    for ho in range(So):
        for wo in range(So):
            g[(2 * ho) * S + 2 * wo, ho * So + wo] = 1.0
    return np.kron(np.eye(B, dtype=np.float32), g)


def _upsample_scatter_np(S, B):
    p = np.zeros((4, S * S, 4 * S * S), np.float32)
    for kh in range(2):
        for kw in range(2):
            t = kh * 2 + kw
            for h in range(S):
                for w in range(S):
                    p[t, h * S + w, (2 * h + kh) * (2 * S) + (2 * w + kw)] = 1.0
    return np.concatenate(
        [np.kron(np.eye(B, dtype=np.float32), p[t]) for t in range(4)], axis=0)


def _conv3x3_relu(xb, S, w_ref, b_ref, m_ref):
    y = jnp.dot(w_ref[...], xb, preferred_element_type=jnp.float32)
    cout = y.shape[0] // 9
    acc = None
    for t, (dh, dw) in enumerate(_TAPS9):
        off = dh * S + dw
        s = y[t * cout:(t + 1) * cout]
        if off:
            s = jnp.roll(s, -off, axis=1)
        if dh != 0 or dw != 0:
            s = s * m_ref[t]
        acc = s if acc is None else acc + s
    return jnp.maximum(acc + b_ref[...], 0.0)


def _maxpool2x2(e, S, g_ref):
    t1 = jnp.roll(e, -1, axis=1)
    t2 = jnp.roll(e, -S, axis=1)
    t3 = jnp.roll(e, -(S + 1), axis=1)
    m = jnp.maximum(jnp.maximum(e, t1), jnp.maximum(t2, t3))
    return jnp.dot(m.astype(_BF), g_ref[...],
                   preferred_element_type=jnp.float32)


def _conv_transpose2x2(xcat, w_ref, b_ref, p_ref, nchunk=1):
    parts = [jnp.dot(w_ref[t], xcat, preferred_element_type=jnp.float32)
             for t in range(4)]
    Lin = xcat.shape[1]
    w = Lin // nchunk
    outs = []
    for c in range(nchunk):
        al = jnp.concatenate([pt[:, c * w:(c + 1) * w] for pt in parts],
                             axis=1).astype(_BF)
        outs.append(jnp.dot(al, p_ref[...], preferred_element_type=jnp.float32))
    o = outs[0] if nchunk == 1 else jnp.concatenate(outs, axis=1)
    return o + b_ref[...]


def _unet_kernel(x_ref, m16, m8, m4, m2,
                 w11, b11, w12, b12, w21, b21, w22, b22,
                 w31, b31, w32, b32, wb1, bb1, wb2, bb2,
                 g1, g2, g3, u3w, u3b, u2w, u2b, u1w, u1b,
                 p2, p4, p8, ow, ob, o_ref):
    C, P = x_ref.shape[1], x_ref.shape[2]

    def both(f):
        return [f(i) for i in range(_CH)]

    x = both(lambda i: jnp.concatenate(
        [x_ref[i * _B + b].astype(_BF) for b in range(_B)], axis=1))

    t = both(lambda i: _conv3x3_relu(x[i], 16, w11, b11, m16).astype(_BF))
    e1 = both(lambda i: _conv3x3_relu(t[i], 16, w12, b12, m16))
    e1b = [v.astype(_BF) for v in e1]
    p1 = both(lambda i: _maxpool2x2(e1[i], 16, g1).astype(_BF))
    t = both(lambda i: _conv3x3_relu(p1[i], 8, w21, b21, m8).astype(_BF))
    e2 = both(lambda i: _conv3x3_relu(t[i], 8, w22, b22, m8))
    e2b = [v.astype(_BF) for v in e2]
    p2v = both(lambda i: _maxpool2x2(e2[i], 8, g2).astype(_BF))
    t = both(lambda i: _conv3x3_relu(p2v[i], 4, w31, b31, m4).astype(_BF))
    e3 = both(lambda i: _conv3x3_relu(t[i], 4, w32, b32, m4))
    e3b = [v.astype(_BF) for v in e3]
    p3v = both(lambda i: _maxpool2x2(e3[i], 4, g3).astype(_BF))
    t = both(lambda i: _conv3x3_relu(p3v[i], 2, wb1, bb1, m2).astype(_BF))
    bn = both(lambda i: _conv3x3_relu(t[i], 2, wb2, bb2, m2))

    u3 = both(lambda i: _conv_transpose2x2(bn[i].astype(_BF), u3w, u3b, p2))
    u2 = both(lambda i: _conv_transpose2x2(
        jnp.concatenate([u3[i].astype(_BF), e3b[i]], axis=0), u2w, u2b, p4))
    u1 = both(lambda i: _conv_transpose2x2(
        jnp.concatenate([u2[i].astype(_BF), e2b[i]], axis=0), u1w, u1b, p8,
        nchunk=4))
    out = both(lambda i: jnp.dot(
        ow[...], jnp.concatenate([u1[i].astype(_BF), e1b[i]], axis=0),
        preferred_element_type=jnp.float32) + ob[...])
    for i in range(_CH):
        for b in range(_B):
            o_ref[i * _B + b] = out[i][:, b * P:(b + 1) * P]


def _w9(w):
    cin, cout = w.shape[2], w.shape[3]
    return jnp.transpose(w, (0, 1, 3, 2)).reshape(9 * cout, cin).astype(_BF)


def _t4(w):
    return jnp.transpose(w, (0, 1, 3, 2)).reshape(4, w.shape[3], w.shape[2]).astype(_BF)


def _col(b):
    return b.reshape(-1, 1)


def kernel(enc1_w1, enc1_b1, enc1_w2, enc1_b2, enc2_w1, enc2_b1, enc2_w2,
           enc2_b2, enc3_w1, enc3_b1, enc3_w2, enc3_b2, bn_w1, bn_b1, bn_w2,
           bn_b2, up3_w, up3_b, up2_w, up2_b, up1_w, up1_b, out_w, out_b, x):
    N, C, H, W = x.shape
    P = H * W
    B = _B
    f32a = lambda a: jnp.asarray(a, dtype=jnp.float32)
    bf = lambda a: jnp.asarray(a, dtype=_BF)

    consts = (
        f32a(_tap_masks_np(16, B)), f32a(_tap_masks_np(8, B)),
        f32a(_tap_masks_np(4, B)), f32a(_tap_masks_np(2, B)),
        _w9(enc1_w1), _col(enc1_b1), _w9(enc1_w2), _col(enc1_b2),
        _w9(enc2_w1), _col(enc2_b1), _w9(enc2_w2), _col(enc2_b2),
        _w9(enc3_w1), _col(enc3_b1), _w9(enc3_w2), _col(enc3_b2),
        _w9(bn_w1), _col(bn_b1), _w9(bn_w2), _col(bn_b2),
        bf(_pool_select_np(16, B)), bf(_pool_select_np(8, B)),
        bf(_pool_select_np(4, B)),
        _t4(up3_w), _col(up3_b), _t4(up2_w), _col(up2_b), _t4(up1_w), _col(up1_b),
        bf(_upsample_scatter_np(2, B)), bf(_upsample_scatter_np(4, B)),
        bf(_upsample_scatter_np(8, 2)),
        out_w.T.astype(_BF), _col(out_b),
    )

    x2 = x.reshape(N, C, P)
    G = _CH * B

    def _call(xs, *cs):
        Ns = xs.shape[0]
        in_specs = [pl.BlockSpec((G, C, P), lambda n: (n, 0, 0))]
        for a in cs:
            in_specs.append(pl.BlockSpec(a.shape, lambda n, _nd=a.ndim: (0,) * _nd))
        return pl.pallas_call(
            _unet_kernel,
            out_shape=jax.ShapeDtypeStruct((Ns, C, P), jnp.float32),
            grid=(Ns // G,),
            in_specs=in_specs,
            out_specs=pl.BlockSpec((G, C, P), lambda n: (n, 0, 0)),
            compiler_params=pltpu.CompilerParams(
                dimension_semantics=("parallel",),
                vmem_limit_bytes=64 * 1024 * 1024),
        )(xs, *cs)

    out = _call(x2, *consts)
    return out.reshape(N, C, H, W)


# speedup vs baseline: 4.5244x; 1.0469x over previous
import numpy as np
import jax
import jax.numpy as jnp
from jax.experimental import pallas as pl
from jax.experimental.pallas import tpu as pltpu

_B = 8
_CH = 8
_TAPS9 = [(dh, dw) for dh in (-1, 0, 1) for dw in (-1, 0, 1)]
_BF = jnp.bfloat16


def _tap_masks_np(S, B):
    P = S * S
    m = np.zeros((9, 1, P), np.float32)
    for t, (dh, dw) in enumerate(_TAPS9):
        for h in range(S):
            for w in range(S):
                if 0 <= h + dh < S and 0 <= w + dw < S:
                    m[t, 0, h * S + w] = 1.0
    return np.tile(m, (1, 1, B))


def _pool_select_np(S, B):
    So = S // 2
    g = np.zeros((S * S, So * So), np.float32)
    for ho in range(So):
        for wo in range(So):
            g[(2 * ho) * S + 2 * wo, ho * So + wo] = 1.0
    return np.kron(np.eye(B, dtype=np.float32), g)


def _upsample_scatter_np(S, B):
    p = np.zeros((4, S * S, 4 * S * S), np.float32)
    for kh in range(2):
        for kw in range(2):
            t = kh * 2 + kw
            for h in range(S):
                for w in range(S):
                    p[t, h * S + w, (2 * h + kh) * (2 * S) + (2 * w + kw)] = 1.0
    return np.concatenate(
        [np.kron(np.eye(B, dtype=np.float32), p[t]) for t in range(4)], axis=0)


def _conv3x3_relu(xb, S, w_all, layer, cin, cout, mslab, ball, bj, m_ref):
    y = jnp.dot(w_all[layer][:, :cin], xb,
                preferred_element_type=jnp.float32)
    acc = None
    for t, (dh, dw) in enumerate(_TAPS9):
        off = dh * S + dw
        s = y[t * mslab:t * mslab + cout]
        if off:
            s = jnp.roll(s, -off, axis=1)
        if dh != 0 or dw != 0:
            s = s * m_ref[t]
        acc = s if acc is None else acc + s
    return jnp.maximum(acc + ball[:cout, bj:bj + 1], 0.0)


def _maxpool2x2(e, S, g_ref):
    t1 = jnp.roll(e, -1, axis=1)
    t2 = jnp.roll(e, -S, axis=1)
    t3 = jnp.roll(e, -(S + 1), axis=1)
    m = jnp.maximum(jnp.maximum(e, t1), jnp.maximum(t2, t3))
    return jnp.dot(m.astype(_BF), g_ref[...],
                   preferred_element_type=jnp.float32)


def _conv_transpose2x2(xcat, ct, layer, cin, cout, ball, bj, p_ref, nchunk=1):
    b_ref = ball[:cout, bj:bj + 1]
    parts = [jnp.dot(ct[layer, t][:cout, :cin], xcat,
                     preferred_element_type=jnp.float32)
             for t in range(4)]
    Lin = xcat.shape[1]
    w = Lin // nchunk
    outs = []
    for c in range(nchunk):
        al = jnp.concatenate([pt[:, c * w:(c + 1) * w] for pt in parts],
                             axis=1).astype(_BF)
        outs.append(jnp.dot(al, p_ref[...], preferred_element_type=jnp.float32))
    o = outs[0] if nchunk == 1 else jnp.concatenate(outs, axis=1)
    return o + b_ref


def _unet_kernel(x_ref, m16, m8, m4, m2, wA, wB, ball,
                 g1, g2, g3, ct, p2, p4, p8, ow, o_ref):
    C, P = x_ref.shape[1], x_ref.shape[2]

    def both(f):
        return [f(i) for i in range(_CH)]

    x = both(lambda i: jnp.concatenate(
        [x_ref[i * _B + b].astype(_BF) for b in range(_B)], axis=1))

    t = both(lambda i: _conv3x3_relu(x[i], 16, wA, 0, 64, 32, 32, ball, 0, m16).astype(_BF))
    e1 = both(lambda i: _conv3x3_relu(t[i], 16, wA, 1, 32, 32, 32, ball, 1, m16))
    e1b = [v.astype(_BF) for v in e1]
    p1 = both(lambda i: _maxpool2x2(e1[i], 16, g1).astype(_BF))
    t = both(lambda i: _conv3x3_relu(p1[i], 8, wB, 0, 32, 16, 16, ball, 2, m8).astype(_BF))
    e2 = both(lambda i: _conv3x3_relu(t[i], 8, wB, 1, 16, 16, 16, ball, 3, m8))
    e2b = [v.astype(_BF) for v in e2]
    p2v = both(lambda i: _maxpool2x2(e2[i], 8, g2).astype(_BF))
    t = both(lambda i: _conv3x3_relu(p2v[i], 4, wB, 2, 16, 8, 16, ball, 4, m4).astype(_BF))
    e3 = both(lambda i: _conv3x3_relu(t[i], 4, wB, 3, 8, 8, 16, ball, 5, m4))
    e3b = [v.astype(_BF) for v in e3]
    p3v = both(lambda i: _maxpool2x2(e3[i], 4, g3).astype(_BF))
    t = both(lambda i: _conv3x3_relu(p3v[i], 2, wB, 4, 8, 8, 16, ball, 6, m2).astype(_BF))
    bn = both(lambda i: _conv3x3_relu(t[i], 2, wB, 5, 8, 8, 16, ball, 7, m2))

    u3 = both(lambda i: _conv_transpose2x2(
        bn[i].astype(_BF), ct, 0, 8, 8, ball, 8, p2))
    u2 = both(lambda i: _conv_transpose2x2(
        jnp.concatenate([u3[i].astype(_BF), e3b[i]], axis=0),
        ct, 1, 16, 16, ball, 9, p4))
    u1 = both(lambda i: _conv_transpose2x2(
        jnp.concatenate([u2[i].astype(_BF), e2b[i]], axis=0),
        ct, 2, 32, 32, ball, 10, p8, nchunk=4))
    out = both(lambda i: jnp.dot(
        ow[...], jnp.concatenate([u1[i].astype(_BF), e1b[i]], axis=0),
        preferred_element_type=jnp.float32) + ball[:, 11:12])
    for i in range(_CH):
        for b in range(_B):
            o_ref[i * _B + b] = out[i][:, b * P:(b + 1) * P]


def _wstack(ws, kpad, mpad):
    padded = [jnp.pad(w, ((0, 0), (0, 0), (0, kpad - w.shape[2]),
                          (0, mpad - w.shape[3]))) for w in ws]
    s = jnp.stack(padded)
    return jnp.transpose(s, (0, 1, 2, 4, 3)).reshape(
        len(ws), 9 * mpad, kpad).astype(_BF)


def _ctstack(ws):
    padded = [jnp.pad(w, ((0, 0), (0, 0), (0, 32 - w.shape[2]),
                          (0, 32 - w.shape[3]))) for w in ws]
    s = jnp.stack(padded)
    return jnp.transpose(s, (0, 1, 2, 4, 3)).reshape(len(ws), 4, 32, 32).astype(_BF)


def _bstack(bs):
    return jnp.stack([jnp.pad(b, (0, 64 - b.shape[0])) for b in bs]).T


def kernel(enc1_w1, enc1_b1, enc1_w2, enc1_b2, enc2_w1, enc2_b1, enc2_w2,
           enc2_b2, enc3_w1, enc3_b1, enc3_w2, enc3_b2, bn_w1, bn_b1, bn_w2,
           bn_b2, up3_w, up3_b, up2_w, up2_b, up1_w, up1_b, out_w, out_b, x):
    N, C, H, W = x.shape
    P = H * W
    B = _B
    f32a = lambda a: jnp.asarray(a, dtype=jnp.float32)
    bf = lambda a: jnp.asarray(a, dtype=_BF)

    consts = (
        f32a(_tap_masks_np(16, B)), f32a(_tap_masks_np(8, B)),
        f32a(_tap_masks_np(4, B)), f32a(_tap_masks_np(2, B)),
        _wstack([enc1_w1, enc1_w2], 64, 32),
        _wstack([enc2_w1, enc2_w2, enc3_w1, enc3_w2, bn_w1, bn_w2], 32, 16),
        _bstack([enc1_b1, enc1_b2, enc2_b1, enc2_b2, enc3_b1, enc3_b2,
                 bn_b1, bn_b2, up3_b, up2_b, up1_b, out_b]),
        bf(_pool_select_np(16, B)), bf(_pool_select_np(8, B)),
        bf(_pool_select_np(4, B)),
        _ctstack([up3_w, up2_w, up1_w]),
        bf(_upsample_scatter_np(2, B)), bf(_upsample_scatter_np(4, B)),
        bf(_upsample_scatter_np(8, 2)),
        out_w.T.astype(_BF),
    )

    x2 = x.reshape(N, C, P)
    G = _CH * B

    def _call(xs, *cs):
        Ns = xs.shape[0]
        in_specs = [pl.BlockSpec((G, C, P), lambda n: (n, 0, 0))]
        for a in cs:
            in_specs.append(pl.BlockSpec(a.shape, lambda n, _nd=a.ndim: (0,) * _nd))
        return pl.pallas_call(
            _unet_kernel,
            out_shape=jax.ShapeDtypeStruct((Ns, C, P), jnp.float32),
            grid=(Ns // G,),
            in_specs=in_specs,
            out_specs=pl.BlockSpec((G, C, P), lambda n: (n, 0, 0)),
            compiler_params=pltpu.CompilerParams(
                dimension_semantics=("parallel",),
                vmem_limit_bytes=64 * 1024 * 1024),
        )(xs, *cs)

    out = _call(x2, *consts)
    return out.reshape(N, C, H, W)
